# baseline (speedup 1.0000x reference)
import functools

import jax
import jax.numpy as jnp
from jax import lax
from jax.experimental import pallas as pl
from jax.experimental.pallas import tpu as pltpu
from jax.experimental.pallas import tpu_sc as plsc

N_FIELDS = 26
VOCAB_P1 = 1001
EMB = 128
BATCH = 16384
DENSE_DIM = 13
OUT_DIM = N_FIELDS * EMB + DENSE_DIM

NC, NS = 2, 16
NW = NC * NS
W = BATCH // NW
B_CHK = 64
NPAIR = W // (2 * B_CHK)
TOTALP = N_FIELDS * NPAIR
L = 16
FH = N_FIELDS // 2
DH = W // 2


def _zero_pad_row(tables):
    def body(t_ref, o_ref):
        row = lax.broadcasted_iota(jnp.int32, (1, VOCAB_P1, EMB), 1)
        o_ref[...] = jnp.where(row == 0, 0.0, t_ref[...])

    return pl.pallas_call(
        body,
        grid=(N_FIELDS,),
        in_specs=[pl.BlockSpec((1, VOCAB_P1, EMB), lambda i: (i, 0, 0))],
        out_specs=pl.BlockSpec((1, VOCAB_P1, EMB), lambda i: (i, 0, 0)),
        out_shape=jax.ShapeDtypeStruct((N_FIELDS, VOCAB_P1, EMB), jnp.float32),
    )(tables)


def _sc_embed_t(feats2, dense, t):
    mesh = plsc.VectorSubcoreMesh(core_axis_name="c", subcore_axis_name="s")

    @functools.partial(
        pl.kernel,
        out_type=jax.ShapeDtypeStruct((OUT_DIM, BATCH), jnp.float32),
        mesh=mesh,
        scratch_types=[
            pltpu.VMEM((FH, W), jnp.int32),
            pltpu.VMEM((4, B_CHK, EMB), jnp.float32),
            pltpu.VMEM((2, B_CHK, EMB), jnp.float32),
            pltpu.VMEM((DH, DENSE_DIM), jnp.float32),
            pltpu.VMEM((DENSE_DIM, DH), jnp.float32),
            pltpu.SemaphoreType.DMA,
            pltpu.SemaphoreType.DMA,
        ],
        compiler_params=pltpu.CompilerParams(needs_layout_passes=False),
    )
    def k(feats_hbm, dense_hbm, t_hbm, out_hbm, idx_v, gbufs, wbufs,
          dense_v, dense_t, gsem, wsem):
        wid = lax.axis_index("c") * NS + lax.axis_index("s")
        base = wid * W

        iota = lax.broadcasted_iota(jnp.int32, (L,), 0)
        rows = [iota + k16 * L for k16 in range(4)]

        def load_idx_half(h):
            pltpu.sync_copy(feats_hbm.at[h].at[:, pl.ds(base, W)], idx_v)

        def gather_start(p, gslot):
            f = p // NPAIR
            jj = p % NPAIR
            f_loc = lax.rem(f, FH)
            for s in range(2):
                pltpu.async_copy(
                    t_hbm.at[f].at[idx_v.at[f_loc,
                                            pl.ds(jj * 2 * B_CHK + s * B_CHK,
                                                  B_CHK)]],
                    gbufs.at[gslot + s], gsem)

        def write_start(p):
            f = p // NPAIR
            jj = p % NPAIR
            for h in range(2):
                pltpu.async_copy(
                    wbufs.at[h],
                    out_hbm.at[pl.ds(f * EMB + h * B_CHK, B_CHK),
                               pl.ds(base + jj * 2 * B_CHK, 2 * B_CHK)], wsem)

        def gather_wait():
            pltpu.make_async_copy(t_hbm.at[0, pl.ds(0, B_CHK)], gbufs.at[0],
                                  gsem).wait()

        def write_wait():
            pltpu.make_async_copy(gbufs.at[0],
                                  out_hbm.at[pl.ds(0, B_CHK),
                                             pl.ds(0, 2 * B_CHK)],
                                  wsem).wait()

        def transpose_quarter(gslot, e_half, wslot, b_half):
            @pl.loop(0, B_CHK, step=2)
            def _(b0):
                for db in range(2):
                    b = b0 + db
                    col = jnp.full((L,), b_half * B_CHK + b, jnp.int32)
                    for k16 in range(4):
                        v = gbufs.at[gslot][b,
                                            pl.ds(e_half * B_CHK + k16 * L, L)]
                        plsc.store_scatter(wbufs.at[wslot], [rows[k16], col], v)

        load_idx_half(0)
        gather_start(0, 0)

        @pl.loop(0, TOTALP // 2)
        def _task(u):
            for par in range(2):
                p = u * 2 + par
                gslot = 2 * par

                @pl.when(p + 1 < TOTALP)
                def _():
                    @pl.when(p + 1 == (TOTALP // 2))
                    def _():
                        load_idx_half(1)
                    gather_start(p + 1, 2 - gslot)

                gather_wait()
                gather_wait()

                @pl.when(p >= 1)
                def _():
                    write_wait()
                    write_wait()

                for h in range(2):
                    for s in range(2):
                        transpose_quarter(gslot + s, h, h, s)
                write_start(p)

        write_wait()
        write_wait()

        for hb in range(2):
            pltpu.sync_copy(dense_hbm.at[pl.ds(base + hb * DH, DH), :],
                            dense_v)
            lane_clamped = jnp.minimum(iota, DENSE_DIM - 1)

            @pl.loop(0, DH, step=2)
            def _(b0):
                for db in range(2):
                    b = b0 + db
                    col = jnp.full((L,), b, jnp.int32)
                    v = plsc.load_gather(dense_v, [col, lane_clamped])
                    plsc.store_scatter(dense_t, [iota, col], v,
                                       mask=iota < DENSE_DIM)

            pltpu.sync_copy(dense_t,
                            out_hbm.at[pl.ds(N_FIELDS * EMB, DENSE_DIM),
                                       pl.ds(base + hb * DH, DH)])

    return k(feats2, dense, t)


def kernel(feat_0, feat_1, feat_2, feat_3, feat_4, feat_5, feat_6, feat_7,
           feat_8, feat_9, feat_10, feat_11, feat_12, feat_13, feat_14,
           feat_15, feat_16, feat_17, feat_18, feat_19, feat_20, feat_21,
           feat_22, feat_23, feat_24, feat_25, dense, tables):
    feats2 = jnp.stack([
        feat_0, feat_1, feat_2, feat_3, feat_4, feat_5, feat_6, feat_7,
        feat_8, feat_9, feat_10, feat_11, feat_12, feat_13, feat_14, feat_15,
        feat_16, feat_17, feat_18, feat_19, feat_20, feat_21, feat_22,
        feat_23, feat_24, feat_25,
    ]).astype(jnp.int32).reshape(2, FH, BATCH)
    t = _zero_pad_row(tables.astype(jnp.float32))
    out_t = _sc_embed_t(feats2, dense.astype(jnp.float32), t)
    return jnp.transpose(out_t)


# speedup vs baseline: 1.0003x; 1.0003x over previous
import functools

import jax
import jax.numpy as jnp
from jax import lax
from jax.experimental import pallas as pl
from jax.experimental.pallas import tpu as pltpu
from jax.experimental.pallas import tpu_sc as plsc

N_FIELDS = 26
VOCAB_P1 = 1001
EMB = 128
BATCH = 16384
DENSE_DIM = 13
OUT_DIM = N_FIELDS * EMB + DENSE_DIM

NC, NS = 2, 16
NW = NC * NS
W = BATCH // NW
B_CHK = 64
NPAIR = W // (2 * B_CHK)
TOTALP = N_FIELDS * NPAIR
L = 16
FH = N_FIELDS // 2
DH = W // 2


def _zero_pad_row(tables):
    def body(t_ref, o_ref):
        row = lax.broadcasted_iota(jnp.int32, (1, VOCAB_P1, EMB), 1)
        o_ref[...] = jnp.where(row == 0, 0.0, t_ref[...])

    return pl.pallas_call(
        body,
        grid=(N_FIELDS,),
        in_specs=[pl.BlockSpec((1, VOCAB_P1, EMB), lambda i: (i, 0, 0))],
        out_specs=pl.BlockSpec((1, VOCAB_P1, EMB), lambda i: (i, 0, 0)),
        out_shape=jax.ShapeDtypeStruct((N_FIELDS, VOCAB_P1, EMB), jnp.float32),
    )(tables)


def _sc_embed_t(feats2, dense, t):
    mesh = plsc.VectorSubcoreMesh(core_axis_name="c", subcore_axis_name="s")

    @functools.partial(
        pl.kernel,
        out_type=jax.ShapeDtypeStruct((OUT_DIM, BATCH), jnp.float32),
        mesh=mesh,
        scratch_types=[
            pltpu.VMEM((FH, W), jnp.int32),
            pltpu.VMEM((4, B_CHK, EMB), jnp.float32),
            pltpu.VMEM((2, B_CHK, EMB + 1), jnp.float32),
            pltpu.VMEM((DH, DENSE_DIM), jnp.float32),
            pltpu.VMEM((DENSE_DIM, DH), jnp.float32),
            pltpu.SemaphoreType.DMA,
            pltpu.SemaphoreType.DMA,
        ],
        compiler_params=pltpu.CompilerParams(needs_layout_passes=False),
    )
    def k(feats_hbm, dense_hbm, t_hbm, out_hbm, idx_v, gbufs, wbufs,
          dense_v, dense_t, gsem, wsem):
        wid = lax.axis_index("c") * NS + lax.axis_index("s")
        base = wid * W

        iota = lax.broadcasted_iota(jnp.int32, (L,), 0)
        rows = [iota + k16 * L for k16 in range(4)]

        def load_idx_half(h):
            pltpu.sync_copy(feats_hbm.at[h].at[:, pl.ds(base, W)], idx_v)

        def gather_start(p, gslot):
            f = p // NPAIR
            jj = p % NPAIR
            f_loc = lax.rem(f, FH)
            for s in range(2):
                pltpu.async_copy(
                    t_hbm.at[f].at[idx_v.at[f_loc,
                                            pl.ds(jj * 2 * B_CHK + s * B_CHK,
                                                  B_CHK)]],
                    gbufs.at[gslot + s], gsem)

        def write_start(p):
            f = p // NPAIR
            jj = p % NPAIR
            for h in range(2):
                pltpu.async_copy(
                    wbufs.at[h].at[:, pl.ds(0, 2 * B_CHK)],
                    out_hbm.at[pl.ds(f * EMB + h * B_CHK, B_CHK),
                               pl.ds(base + jj * 2 * B_CHK, 2 * B_CHK)], wsem)

        def gather_wait():
            pltpu.make_async_copy(t_hbm.at[0, pl.ds(0, B_CHK)], gbufs.at[0],
                                  gsem).wait()

        def write_wait():
            pltpu.make_async_copy(gbufs.at[0],
                                  out_hbm.at[pl.ds(0, B_CHK),
                                             pl.ds(0, 2 * B_CHK)],
                                  wsem).wait()

        def transpose_quarter(gslot, e_half, wslot, b_half):
            @pl.loop(0, B_CHK, step=2)
            def _(b0):
                for db in range(2):
                    b = b0 + db
                    col = jnp.full((L,), b_half * B_CHK + b, jnp.int32)
                    for k16 in range(4):
                        v = gbufs.at[gslot][b,
                                            pl.ds(e_half * B_CHK + k16 * L, L)]
                        plsc.store_scatter(wbufs.at[wslot], [rows[k16], col], v)

        load_idx_half(0)
        gather_start(0, 0)

        @pl.loop(0, TOTALP // 2)
        def _task(u):
            for par in range(2):
                p = u * 2 + par
                gslot = 2 * par

                @pl.when(p + 1 < TOTALP)
                def _():
                    @pl.when(p + 1 == (TOTALP // 2))
                    def _():
                        load_idx_half(1)
                    gather_start(p + 1, 2 - gslot)

                gather_wait()
                gather_wait()

                @pl.when(p >= 1)
                def _():
                    write_wait()
                    write_wait()

                for h in range(2):
                    for s in range(2):
                        transpose_quarter(gslot + s, h, h, s)
                write_start(p)

        write_wait()
        write_wait()

        for hb in range(2):
            pltpu.sync_copy(dense_hbm.at[pl.ds(base + hb * DH, DH), :],
                            dense_v)
            lane_clamped = jnp.minimum(iota, DENSE_DIM - 1)

            @pl.loop(0, DH, step=2)
            def _(b0):
                for db in range(2):
                    b = b0 + db
                    col = jnp.full((L,), b, jnp.int32)
                    v = plsc.load_gather(dense_v, [col, lane_clamped])
                    plsc.store_scatter(dense_t, [iota, col], v,
                                       mask=iota < DENSE_DIM)

            pltpu.sync_copy(dense_t,
                            out_hbm.at[pl.ds(N_FIELDS * EMB, DENSE_DIM),
                                       pl.ds(base + hb * DH, DH)])

    return k(feats2, dense, t)


def kernel(feat_0, feat_1, feat_2, feat_3, feat_4, feat_5, feat_6, feat_7,
           feat_8, feat_9, feat_10, feat_11, feat_12, feat_13, feat_14,
           feat_15, feat_16, feat_17, feat_18, feat_19, feat_20, feat_21,
           feat_22, feat_23, feat_24, feat_25, dense, tables):
    feats2 = jnp.stack([
        feat_0, feat_1, feat_2, feat_3, feat_4, feat_5, feat_6, feat_7,
        feat_8, feat_9, feat_10, feat_11, feat_12, feat_13, feat_14, feat_15,
        feat_16, feat_17, feat_18, feat_19, feat_20, feat_21, feat_22,
        feat_23, feat_24, feat_25,
    ]).astype(jnp.int32).reshape(2, FH, BATCH)
    t = _zero_pad_row(tables.astype(jnp.float32))
    out_t = _sc_embed_t(feats2, dense.astype(jnp.float32), t)
    return jnp.transpose(out_t)


# speedup vs baseline: 3.7291x; 3.7279x over previous
import functools

import jax
import jax.numpy as jnp
from jax import lax
from jax.experimental import pallas as pl
from jax.experimental.pallas import tpu as pltpu
from jax.experimental.pallas import tpu_sc as plsc

N_FIELDS = 26
VOCAB_P1 = 1001
EMB = 128
BATCH = 16384
DENSE_DIM = 13
OUT_DIM = N_FIELDS * EMB + DENSE_DIM

NC, NS = 2, 16
NW = NC * NS
W = BATCH // NW
B_CHK = 64
NPAIR = W // (2 * B_CHK)
TOTALP = N_FIELDS * NPAIR
L = 16
FH = N_FIELDS // 2
DH = W // 2


def _zero_pad_row(tables):
    def body(t_ref, o_ref):
        row = lax.broadcasted_iota(jnp.int32, (1, VOCAB_P1, EMB), 1)
        o_ref[...] = jnp.where(row == 0, 0.0, t_ref[...])

    return pl.pallas_call(
        body,
        grid=(N_FIELDS,),
        in_specs=[pl.BlockSpec((1, VOCAB_P1, EMB), lambda i: (i, 0, 0))],
        out_specs=pl.BlockSpec((1, VOCAB_P1, EMB), lambda i: (i, 0, 0)),
        out_shape=jax.ShapeDtypeStruct((N_FIELDS, VOCAB_P1, EMB), jnp.float32),
    )(tables)


def _sc_embed_t(feats2, dense, t):
    mesh = plsc.VectorSubcoreMesh(core_axis_name="c", subcore_axis_name="s")

    @functools.partial(
        pl.kernel,
        out_type=jax.ShapeDtypeStruct((OUT_DIM, BATCH), jnp.float32),
        mesh=mesh,
        scratch_types=[
            pltpu.VMEM((FH, W), jnp.int32),
            pltpu.VMEM((4, B_CHK, EMB), jnp.float32),
            pltpu.VMEM((2, B_CHK, 2 * B_CHK), jnp.float32),
            pltpu.VMEM((DH, DENSE_DIM), jnp.float32),
            pltpu.VMEM((DENSE_DIM, DH), jnp.float32),
            pltpu.VMEM((L, L), jnp.int32),
            pltpu.SemaphoreType.DMA,
            pltpu.SemaphoreType.DMA,
        ],
        compiler_params=pltpu.CompilerParams(needs_layout_passes=False),
    )
    def k(feats_hbm, dense_hbm, t_hbm, out_hbm, idx_v, gbufs, wbufs,
          dense_v, dense_t, dtab, gsem, wsem):
        wid = lax.axis_index("c") * NS + lax.axis_index("s")
        base = wid * W

        iota = lax.broadcasted_iota(jnp.int32, (L,), 0)
        for d in range(L):
            dtab[d, pl.ds(0, L)] = lax.rem(iota + d, L)
        cvec = [iota + q * L for q in range(8)]

        def load_idx_half(h):
            pltpu.sync_copy(feats_hbm.at[h].at[:, pl.ds(base, W)], idx_v)

        def gather_start(p, gslot):
            f = p // NPAIR
            jj = p % NPAIR
            f_loc = lax.rem(f, FH)
            for s in range(2):
                pltpu.async_copy(
                    t_hbm.at[f].at[idx_v.at[f_loc,
                                            pl.ds(jj * 2 * B_CHK + s * B_CHK,
                                                  B_CHK)]],
                    gbufs.at[gslot + s], gsem)

        def write_start(p):
            f = p // NPAIR
            jj = p % NPAIR
            for h in range(2):
                pltpu.async_copy(
                    wbufs.at[h],
                    out_hbm.at[pl.ds(f * EMB + h * B_CHK, B_CHK),
                               pl.ds(base + jj * 2 * B_CHK, 2 * B_CHK)], wsem)

        def gather_wait():
            pltpu.make_async_copy(t_hbm.at[0, pl.ds(0, B_CHK)], gbufs.at[0],
                                  gsem).wait()

        def write_wait():
            pltpu.make_async_copy(gbufs.at[0],
                                  out_hbm.at[pl.ds(0, B_CHK),
                                             pl.ds(0, 2 * B_CHK)],
                                  wsem).wait()

        def transpose_pair(gslot):
            @pl.loop(0, L)
            def _(d):
                dm = dtab[d, pl.ds(0, L)]
                for h in range(2):
                    for s in range(2):
                        for eq in range(4):
                            cols_l = dm + (h * B_CHK + eq * L)
                            rows_w = dm + (eq * L)
                            vs = [plsc.load_gather(gbufs.at[gslot + s],
                                                   [cvec[bq], cols_l])
                                  for bq in range(4)]
                            for bq in range(4):
                                plsc.store_scatter(
                                    wbufs.at[h],
                                    [rows_w, cvec[s * 4 + bq]], vs[bq])

        load_idx_half(0)
        gather_start(0, 0)

        @pl.loop(0, TOTALP // 2)
        def _task(u):
            for par in range(2):
                p = u * 2 + par
                gslot = 2 * par

                @pl.when(p + 1 < TOTALP)
                def _():
                    @pl.when(p + 1 == (TOTALP // 2))
                    def _():
                        load_idx_half(1)
                    gather_start(p + 1, 2 - gslot)

                gather_wait()
                gather_wait()

                @pl.when(p >= 1)
                def _():
                    write_wait()
                    write_wait()

                transpose_pair(gslot)
                write_start(p)

        write_wait()
        write_wait()

        for hb in range(2):
            pltpu.sync_copy(dense_hbm.at[pl.ds(base + hb * DH, DH), :],
                            dense_v)
            lane_clamped = jnp.minimum(iota, DENSE_DIM - 1)

            @pl.loop(0, DH, step=2)
            def _(b0):
                for db in range(2):
                    b = b0 + db
                    col = jnp.full((L,), b, jnp.int32)
                    v = plsc.load_gather(dense_v, [col, lane_clamped])
                    plsc.store_scatter(dense_t, [iota, col], v,
                                       mask=iota < DENSE_DIM)

            pltpu.sync_copy(dense_t,
                            out_hbm.at[pl.ds(N_FIELDS * EMB, DENSE_DIM),
                                       pl.ds(base + hb * DH, DH)])

    return k(feats2, dense, t)


def kernel(feat_0, feat_1, feat_2, feat_3, feat_4, feat_5, feat_6, feat_7,
           feat_8, feat_9, feat_10, feat_11, feat_12, feat_13, feat_14,
           feat_15, feat_16, feat_17, feat_18, feat_19, feat_20, feat_21,
           feat_22, feat_23, feat_24, feat_25, dense, tables):
    feats2 = jnp.stack([
        feat_0, feat_1, feat_2, feat_3, feat_4, feat_5, feat_6, feat_7,
        feat_8, feat_9, feat_10, feat_11, feat_12, feat_13, feat_14, feat_15,
        feat_16, feat_17, feat_18, feat_19, feat_20, feat_21, feat_22,
        feat_23, feat_24, feat_25,
    ]).astype(jnp.int32).reshape(2, FH, BATCH)
    t = _zero_pad_row(tables.astype(jnp.float32))
    out_t = _sc_embed_t(feats2, dense.astype(jnp.float32), t)
    return jnp.transpose(out_t)


# speedup vs baseline: 3.7454x; 1.0044x over previous
import functools

import jax
import jax.numpy as jnp
from jax import lax
from jax.experimental import pallas as pl
from jax.experimental.pallas import tpu as pltpu
from jax.experimental.pallas import tpu_sc as plsc

N_FIELDS = 26
VOCAB_P1 = 1001
EMB = 128
BATCH = 16384
DENSE_DIM = 13
OUT_DIM = N_FIELDS * EMB + DENSE_DIM

NC, NS = 2, 16
NW = NC * NS
W = BATCH // NW
B_CHK = 64
NPAIR = W // (2 * B_CHK)
TOTALP = N_FIELDS * NPAIR
L = 16
FH = N_FIELDS // 2
DH = W // 2


def _zero_pad_row(tables):
    def body(t_ref, o_ref):
        o_ref[...] = t_ref[...]
        o_ref[:, 0:1, :] = jnp.zeros((1, 1, EMB), jnp.float32)

    return pl.pallas_call(
        body,
        grid=(N_FIELDS,),
        in_specs=[pl.BlockSpec((1, VOCAB_P1, EMB), lambda i: (i, 0, 0))],
        out_specs=pl.BlockSpec((1, VOCAB_P1, EMB), lambda i: (i, 0, 0)),
        out_shape=jax.ShapeDtypeStruct((N_FIELDS, VOCAB_P1, EMB), jnp.float32),
    )(tables)


def _dense_transpose(dense):
    def body(d_ref, o_ref):
        o_ref[...] = jnp.transpose(d_ref[...], (1, 0))

    return pl.pallas_call(
        body,
        grid=(NW,),
        in_specs=[pl.BlockSpec((W, DENSE_DIM), lambda i: (i, 0))],
        out_specs=pl.BlockSpec((DENSE_DIM, W), lambda i: (0, i)),
        out_shape=jax.ShapeDtypeStruct((DENSE_DIM, BATCH), jnp.float32),
    )(dense)


def _sc_embed_t(feats2, dense_t_hbm, t):
    mesh = plsc.VectorSubcoreMesh(core_axis_name="c", subcore_axis_name="s")

    @functools.partial(
        pl.kernel,
        out_type=jax.ShapeDtypeStruct((OUT_DIM, BATCH), jnp.float32),
        mesh=mesh,
        scratch_types=[
            pltpu.VMEM((FH, W), jnp.int32),
            pltpu.VMEM((4, B_CHK, EMB), jnp.float32),
            pltpu.VMEM((4, B_CHK, 2 * B_CHK), jnp.float32),
            pltpu.VMEM((L, L), jnp.int32),
            pltpu.SemaphoreType.DMA,
            pltpu.SemaphoreType.DMA,
        ],
        compiler_params=pltpu.CompilerParams(needs_layout_passes=False),
    )
    def k(feats_hbm, dense_hbm, t_hbm, out_hbm, idx_v, gbufs, wbufs,
          dtab, gsem, wsem):
        wid = lax.axis_index("c") * NS + lax.axis_index("s")
        base = wid * W

        iota = lax.broadcasted_iota(jnp.int32, (L,), 0)
        for d in range(L):
            dtab[d, pl.ds(0, L)] = lax.rem(iota + d, L)
        cvec = [iota + q * L for q in range(8)]

        def load_idx_half(h):
            pltpu.sync_copy(feats_hbm.at[h].at[:, pl.ds(base, W)], idx_v)

        def gather_start(p, gslot):
            f = p // NPAIR
            jj = p % NPAIR
            f_loc = lax.rem(f, FH)
            for s in range(2):
                pltpu.async_copy(
                    t_hbm.at[f].at[idx_v.at[f_loc,
                                            pl.ds(jj * 2 * B_CHK + s * B_CHK,
                                                  B_CHK)]],
                    gbufs.at[gslot + s], gsem)

        def write_start(p, wbase):
            f = p // NPAIR
            jj = p % NPAIR
            for h in range(2):
                pltpu.async_copy(
                    wbufs.at[wbase + h],
                    out_hbm.at[pl.ds(f * EMB + h * B_CHK, B_CHK),
                               pl.ds(base + jj * 2 * B_CHK, 2 * B_CHK)], wsem)

        def gather_wait():
            pltpu.make_async_copy(t_hbm.at[0, pl.ds(0, B_CHK)], gbufs.at[0],
                                  gsem).wait()

        def write_wait():
            pltpu.make_async_copy(gbufs.at[0],
                                  out_hbm.at[pl.ds(0, B_CHK),
                                             pl.ds(0, 2 * B_CHK)],
                                  wsem).wait()

        def transpose_pair(gslot, wbase):
            @pl.loop(0, L)
            def _(d):
                dm = dtab[d, pl.ds(0, L)]
                for h in range(2):
                    for s in range(2):
                        for eq in range(4):
                            cols_l = dm + (h * B_CHK + eq * L)
                            rows_w = dm + (eq * L)
                            vs = [plsc.load_gather(gbufs.at[gslot + s],
                                                   [cvec[bq], cols_l])
                                  for bq in range(4)]
                            for bq in range(4):
                                plsc.store_scatter(
                                    wbufs.at[wbase + h],
                                    [rows_w, cvec[s * 4 + bq]], vs[bq])

        load_idx_half(0)
        gather_start(0, 0)

        @pl.loop(0, TOTALP // 2)
        def _task(u):
            for par in range(2):
                p = u * 2 + par
                gslot = 2 * par

                @pl.when(p + 1 < TOTALP)
                def _():
                    @pl.when(p + 1 == (TOTALP // 2))
                    def _():
                        load_idx_half(1)
                    gather_start(p + 1, 2 - gslot)

                gather_wait()
                gather_wait()

                @pl.when(p >= 2)
                def _():
                    write_wait()
                    write_wait()

                transpose_pair(gslot, gslot)
                write_start(p, gslot)

        write_wait()
        write_wait()
        write_wait()
        write_wait()

        pltpu.sync_copy(dense_hbm.at[:, pl.ds(base, W)],
                        out_hbm.at[pl.ds(N_FIELDS * EMB, DENSE_DIM),
                                   pl.ds(base, W)])

    return k(feats2, dense_t_hbm, t)


def kernel(feat_0, feat_1, feat_2, feat_3, feat_4, feat_5, feat_6, feat_7,
           feat_8, feat_9, feat_10, feat_11, feat_12, feat_13, feat_14,
           feat_15, feat_16, feat_17, feat_18, feat_19, feat_20, feat_21,
           feat_22, feat_23, feat_24, feat_25, dense, tables):
    feats2 = jnp.stack([
        feat_0, feat_1, feat_2, feat_3, feat_4, feat_5, feat_6, feat_7,
        feat_8, feat_9, feat_10, feat_11, feat_12, feat_13, feat_14, feat_15,
        feat_16, feat_17, feat_18, feat_19, feat_20, feat_21, feat_22,
        feat_23, feat_24, feat_25,
    ]).astype(jnp.int32).reshape(2, FH, BATCH)
    t = _zero_pad_row(tables.astype(jnp.float32))
    dense_t = _dense_transpose(dense.astype(jnp.float32))
    out_t = _sc_embed_t(feats2, dense_t, t)
    return jnp.transpose(out_t)


# speedup vs baseline: 3.8796x; 1.0358x over previous
import functools

import jax
import jax.numpy as jnp
from jax import lax
from jax.experimental import pallas as pl
from jax.experimental.pallas import tpu as pltpu
from jax.experimental.pallas import tpu_sc as plsc

N_FIELDS = 26
VOCAB_P1 = 1001
EMB = 128
BATCH = 16384
DENSE_DIM = 13
OUT_DIM = N_FIELDS * EMB + DENSE_DIM

NC, NS = 2, 16
NW = NC * NS
W = BATCH // NW
B_CHK = 64
NPAIR = W // (2 * B_CHK)
TOTALP = N_FIELDS * NPAIR
L = 16
FH = N_FIELDS // 2
DH = W // 2


def _dense_transpose(dense):
    def body(d_ref, o_ref):
        o_ref[...] = jnp.transpose(d_ref[...], (1, 0))

    return pl.pallas_call(
        body,
        grid=(NW,),
        in_specs=[pl.BlockSpec((W, DENSE_DIM), lambda i: (i, 0))],
        out_specs=pl.BlockSpec((DENSE_DIM, W), lambda i: (0, i)),
        out_shape=jax.ShapeDtypeStruct((DENSE_DIM, BATCH), jnp.float32),
    )(dense)


def _sc_embed_t(feats2, dense_t_hbm, t):
    mesh = plsc.VectorSubcoreMesh(core_axis_name="c", subcore_axis_name="s")

    @functools.partial(
        pl.kernel,
        out_type=jax.ShapeDtypeStruct((OUT_DIM, BATCH), jnp.float32),
        mesh=mesh,
        scratch_types=[
            pltpu.VMEM((FH, W), jnp.int32),
            pltpu.VMEM((4, B_CHK, EMB), jnp.float32),
            pltpu.VMEM((4, B_CHK, 2 * B_CHK), jnp.float32),
            pltpu.VMEM((L, L), jnp.int32),
            pltpu.SemaphoreType.DMA,
            pltpu.SemaphoreType.DMA,
        ],
        compiler_params=pltpu.CompilerParams(needs_layout_passes=False),
    )
    def k(feats_hbm, dense_hbm, t_hbm, out_hbm, idx_v, gbufs, wbufs,
          dtab, gsem, wsem):
        wid = lax.axis_index("c") * NS + lax.axis_index("s")
        base = wid * W

        iota = lax.broadcasted_iota(jnp.int32, (L,), 0)
        for d in range(L):
            dtab[d, pl.ds(0, L)] = lax.rem(iota + d, L)
        cvec = [iota + q * L for q in range(8)]
        zeros16 = jnp.zeros((L,), jnp.float32)

        def load_idx_half(h):
            pltpu.sync_copy(feats_hbm.at[h].at[:, pl.ds(base, W)], idx_v)

        def gather_start(p, gslot):
            f = p // NPAIR
            jj = p % NPAIR
            f_loc = lax.rem(f, FH)
            for s in range(2):
                pltpu.async_copy(
                    t_hbm.at[f].at[idx_v.at[f_loc,
                                            pl.ds(jj * 2 * B_CHK + s * B_CHK,
                                                  B_CHK)]],
                    gbufs.at[gslot + s], gsem)

        def write_start(p, wbase):
            f = p // NPAIR
            jj = p % NPAIR
            for h in range(2):
                pltpu.async_copy(
                    wbufs.at[wbase + h],
                    out_hbm.at[pl.ds(f * EMB + h * B_CHK, B_CHK),
                               pl.ds(base + jj * 2 * B_CHK, 2 * B_CHK)], wsem)

        def gather_wait():
            pltpu.make_async_copy(t_hbm.at[0, pl.ds(0, B_CHK)], gbufs.at[0],
                                  gsem).wait()

        def write_wait():
            pltpu.make_async_copy(gbufs.at[0],
                                  out_hbm.at[pl.ds(0, B_CHK),
                                             pl.ds(0, 2 * B_CHK)],
                                  wsem).wait()

        def transpose_pair(gslot, wbase):
            @pl.loop(0, L)
            def _(d):
                dm = dtab[d, pl.ds(0, L)]
                for h in range(2):
                    for s in range(2):
                        for eq in range(4):
                            cols_l = dm + (h * B_CHK + eq * L)
                            rows_w = dm + (eq * L)
                            vs = [plsc.load_gather(gbufs.at[gslot + s],
                                                   [cvec[bq], cols_l])
                                  for bq in range(4)]
                            for bq in range(4):
                                plsc.store_scatter(
                                    wbufs.at[wbase + h],
                                    [rows_w, cvec[s * 4 + bq]], vs[bq])

        load_idx_half(0)
        gather_start(0, 0)

        @pl.loop(0, TOTALP // 2)
        def _task(u):
            for par in range(2):
                p = u * 2 + par
                gslot = 2 * par

                @pl.when(p + 1 < TOTALP)
                def _():
                    @pl.when(p + 1 == (TOTALP // 2))
                    def _():
                        load_idx_half(1)
                    gather_start(p + 1, 2 - gslot)

                gather_wait()
                gather_wait()

                @pl.when(p >= 2)
                def _():
                    write_wait()
                    write_wait()

                transpose_pair(gslot, gslot)

                f = p // NPAIR
                jj = p % NPAIR
                f_loc = lax.rem(f, FH)
                for c in range(8):
                    chunk = idx_v[f_loc, pl.ds(jj * 2 * B_CHK + c * L, L)]

                    @pl.when(jnp.min(chunk) == 0)
                    def _():
                        m = chunk == 0

                        @pl.loop(0, B_CHK)
                        def _(e):
                            er = jnp.full((L,), e, jnp.int32)
                            for h in range(2):
                                plsc.store_scatter(wbufs.at[gslot + h],
                                                   [er, cvec[c]], zeros16,
                                                   mask=m)

                write_start(p, gslot)

        write_wait()
        write_wait()
        write_wait()
        write_wait()

        pltpu.sync_copy(dense_hbm.at[:, pl.ds(base, W)],
                        out_hbm.at[pl.ds(N_FIELDS * EMB, DENSE_DIM),
                                   pl.ds(base, W)])

    return k(feats2, dense_t_hbm, t)


def kernel(feat_0, feat_1, feat_2, feat_3, feat_4, feat_5, feat_6, feat_7,
           feat_8, feat_9, feat_10, feat_11, feat_12, feat_13, feat_14,
           feat_15, feat_16, feat_17, feat_18, feat_19, feat_20, feat_21,
           feat_22, feat_23, feat_24, feat_25, dense, tables):
    feats2 = jnp.stack([
        feat_0, feat_1, feat_2, feat_3, feat_4, feat_5, feat_6, feat_7,
        feat_8, feat_9, feat_10, feat_11, feat_12, feat_13, feat_14, feat_15,
        feat_16, feat_17, feat_18, feat_19, feat_20, feat_21, feat_22,
        feat_23, feat_24, feat_25,
    ]).astype(jnp.int32).reshape(2, FH, BATCH)
    dense_t = _dense_transpose(dense.astype(jnp.float32))
    out_t = _sc_embed_t(feats2, dense_t, tables.astype(jnp.float32))
    return jnp.transpose(out_t)


# speedup vs baseline: 3.9302x; 1.0131x over previous
import functools

import jax
import jax.numpy as jnp
from jax import lax
from jax.experimental import pallas as pl
from jax.experimental.pallas import tpu as pltpu
from jax.experimental.pallas import tpu_sc as plsc

N_FIELDS = 26
VOCAB_P1 = 1001
EMB = 128
BATCH = 16384
DENSE_DIM = 13
OUT_DIM = N_FIELDS * EMB + DENSE_DIM

NC, NS = 2, 16
NW = NC * NS
W = BATCH // NW
B_CHK = 64
NPAIR = W // (2 * B_CHK)
TOTALP = N_FIELDS * NPAIR
L = 16
FH = N_FIELDS // 2
DH = W // 2


def _dense_transpose(dense):
    def body(d_ref, o_ref):
        o_ref[...] = jnp.transpose(d_ref[...], (1, 0))

    return pl.pallas_call(
        body,
        grid=(NW,),
        in_specs=[pl.BlockSpec((W, DENSE_DIM), lambda i: (i, 0))],
        out_specs=pl.BlockSpec((DENSE_DIM, W), lambda i: (0, i)),
        out_shape=jax.ShapeDtypeStruct((DENSE_DIM, BATCH), jnp.float32),
    )(dense)


def _sc_embed_t(feats2, dense_t_hbm, t):
    mesh = plsc.VectorSubcoreMesh(core_axis_name="c", subcore_axis_name="s")

    @functools.partial(
        pl.kernel,
        out_type=jax.ShapeDtypeStruct((OUT_DIM, BATCH), jnp.float32),
        mesh=mesh,
        scratch_types=[
            pltpu.VMEM((FH, W), jnp.int32),
            pltpu.VMEM((4, B_CHK, EMB), jnp.float32),
            pltpu.VMEM((4, B_CHK, 2 * B_CHK), jnp.float32),
            pltpu.VMEM((L, L), jnp.int32),
            pltpu.SemaphoreType.DMA,
            pltpu.SemaphoreType.DMA,
        ],
        compiler_params=pltpu.CompilerParams(needs_layout_passes=False),
    )
    def k(feats_hbm, dense_hbm, t_hbm, out_hbm, idx_v, gbufs, wbufs,
          dtab, gsem, wsem):
        wid = lax.axis_index("c") * NS + lax.axis_index("s")
        base = wid * W

        iota = lax.broadcasted_iota(jnp.int32, (L,), 0)
        for d in range(L):
            dtab[d, pl.ds(0, L)] = lax.rem(iota + d, L)
        cvec = [iota + q * L for q in range(8)]
        zeros16 = jnp.zeros((L,), jnp.float32)

        def load_idx_half(h):
            pltpu.sync_copy(feats_hbm.at[h].at[:, pl.ds(base, W)], idx_v)

        def gather_start(p, gslot):
            f = p // NPAIR
            jj = p % NPAIR
            f_loc = lax.rem(f, FH)
            for s in range(2):
                pltpu.async_copy(
                    t_hbm.at[f].at[idx_v.at[f_loc,
                                            pl.ds(jj * 2 * B_CHK + s * B_CHK,
                                                  B_CHK)]],
                    gbufs.at[gslot + s], gsem)

        def write_start(p, wbase):
            f = p // NPAIR
            jj = p % NPAIR
            for h in range(2):
                pltpu.async_copy(
                    wbufs.at[wbase + h],
                    out_hbm.at[pl.ds(f * EMB + h * B_CHK, B_CHK),
                               pl.ds(base + jj * 2 * B_CHK, 2 * B_CHK)], wsem)

        def gather_wait():
            pltpu.make_async_copy(t_hbm.at[0, pl.ds(0, B_CHK)], gbufs.at[0],
                                  gsem).wait()

        def write_wait():
            pltpu.make_async_copy(gbufs.at[0],
                                  out_hbm.at[pl.ds(0, B_CHK),
                                             pl.ds(0, 2 * B_CHK)],
                                  wsem).wait()

        def transpose_pair(gslot, wbase):
            @pl.loop(0, L)
            def _(d):
                dm = dtab[d, pl.ds(0, L)]
                for h in range(2):
                    for s in range(2):
                        for eq in range(4):
                            cols_l = dm + (h * B_CHK + eq * L)
                            rows_w = dm + (eq * L)
                            vs = [plsc.load_gather(gbufs.at[gslot + s],
                                                   [cvec[bq], cols_l])
                                  for bq in range(4)]
                            for bq in range(4):
                                plsc.store_scatter(
                                    wbufs.at[wbase + h],
                                    [rows_w, cvec[s * 4 + bq]], vs[bq])

        load_idx_half(0)
        gather_start(0, 0)

        @pl.loop(0, TOTALP // 2)
        def _task(u):
            for par in range(2):
                p = u * 2 + par
                gslot = 2 * par
                f = p // NPAIR
                jj = p % NPAIR
                f_loc = lax.rem(f, FH)
                chunks = [idx_v[f_loc, pl.ds(jj * 2 * B_CHK + c * L, L)]
                          for c in range(8)]

                @pl.when(p + 1 < TOTALP)
                def _():
                    @pl.when(p + 1 == (TOTALP // 2))
                    def _():
                        load_idx_half(1)
                    gather_start(p + 1, 2 - gslot)

                gather_wait()
                gather_wait()

                @pl.when(p >= 2)
                def _():
                    write_wait()
                    write_wait()

                transpose_pair(gslot, gslot)

                for c in range(8):
                    chunk = chunks[c]

                    @pl.when(jnp.min(chunk) == 0)
                    def _():
                        m = chunk == 0

                        @pl.loop(0, B_CHK)
                        def _(e):
                            er = jnp.full((L,), e, jnp.int32)
                            for h in range(2):
                                plsc.store_scatter(wbufs.at[gslot + h],
                                                   [er, cvec[c]], zeros16,
                                                   mask=m)

                write_start(p, gslot)

        write_wait()
        write_wait()
        write_wait()
        write_wait()

        pltpu.sync_copy(dense_hbm.at[:, pl.ds(base, W)],
                        out_hbm.at[pl.ds(N_FIELDS * EMB, DENSE_DIM),
                                   pl.ds(base, W)])

    return k(feats2, dense_t_hbm, t)


def kernel(feat_0, feat_1, feat_2, feat_3, feat_4, feat_5, feat_6, feat_7,
           feat_8, feat_9, feat_10, feat_11, feat_12, feat_13, feat_14,
           feat_15, feat_16, feat_17, feat_18, feat_19, feat_20, feat_21,
           feat_22, feat_23, feat_24, feat_25, dense, tables):
    feats2 = jnp.stack([
        feat_0, feat_1, feat_2, feat_3, feat_4, feat_5, feat_6, feat_7,
        feat_8, feat_9, feat_10, feat_11, feat_12, feat_13, feat_14, feat_15,
        feat_16, feat_17, feat_18, feat_19, feat_20, feat_21, feat_22,
        feat_23, feat_24, feat_25,
    ]).astype(jnp.int32).reshape(2, FH, BATCH)
    dense_t = _dense_transpose(dense.astype(jnp.float32))
    out_t = _sc_embed_t(feats2, dense_t, tables.astype(jnp.float32))
    return jnp.transpose(out_t)


# speedup vs baseline: 4.1311x; 1.0511x over previous
import functools

import jax
import jax.numpy as jnp
from jax import lax
from jax.experimental import pallas as pl
from jax.experimental.pallas import tpu as pltpu
from jax.experimental.pallas import tpu_sc as plsc

N_FIELDS = 26
VOCAB_P1 = 1001
EMB = 128
BATCH = 16384
DENSE_DIM = 13
OUT_DIM = N_FIELDS * EMB + DENSE_DIM

NC, NS = 2, 16
NW = NC * NS
W = BATCH // NW
B_CHK = 64
NPAIR = W // (2 * B_CHK)
TOTALP = N_FIELDS * NPAIR
L = 16
FH = N_FIELDS // 2
DH = W // 2


def _dense_transpose(dense):
    def body(d_ref, o_ref):
        o_ref[...] = jnp.transpose(d_ref[...], (1, 0))

    return pl.pallas_call(
        body,
        grid=(8,),
        in_specs=[pl.BlockSpec((BATCH // 8, DENSE_DIM), lambda i: (i, 0))],
        out_specs=pl.BlockSpec((DENSE_DIM, BATCH // 8), lambda i: (0, i)),
        out_shape=jax.ShapeDtypeStruct((DENSE_DIM, BATCH), jnp.float32),
    )(dense)


def _sc_embed_t(feats_flat, dense_t_hbm, t):
    mesh = plsc.VectorSubcoreMesh(core_axis_name="c", subcore_axis_name="s")

    @functools.partial(
        pl.kernel,
        out_type=jax.ShapeDtypeStruct((OUT_DIM, BATCH), jnp.float32),
        mesh=mesh,
        scratch_types=[
            pltpu.VMEM((FH * W,), jnp.int32),
            pltpu.VMEM((4, B_CHK, EMB), jnp.float32),
            pltpu.VMEM((4, B_CHK, 2 * B_CHK), jnp.float32),
            pltpu.VMEM((L, L), jnp.int32),
            pltpu.SemaphoreType.DMA,
            pltpu.SemaphoreType.DMA,
            pltpu.SemaphoreType.DMA,
        ],
        compiler_params=pltpu.CompilerParams(needs_layout_passes=False),
    )
    def k(feats_hbm, dense_hbm, t_hbm, out_hbm, idx_v, gbufs, wbufs,
          dtab, gsem, wsem, isem):
        wid = lax.axis_index("c") * NS + lax.axis_index("s")
        base = wid * W

        iota = lax.broadcasted_iota(jnp.int32, (L,), 0)
        for d in range(L):
            dtab[d, pl.ds(0, L)] = lax.rem(iota + d, L)
        cvec = [iota + q * L for q in range(8)]
        zeros16 = jnp.zeros((L,), jnp.float32)

        def load_idx_half(h):
            for fl in range(FH):
                pltpu.async_copy(
                    feats_hbm.at[pl.ds((h * FH + fl) * BATCH + base, W)],
                    idx_v.at[pl.ds(fl * W, W)], isem)
            for fl in range(FH):
                pltpu.make_async_copy(feats_hbm.at[pl.ds(0, W)],
                                      idx_v.at[pl.ds(0, W)], isem).wait()

        def gather_start(p, gslot):
            f = p // NPAIR
            jj = p % NPAIR
            f_loc = lax.rem(f, FH)
            for s in range(2):
                pltpu.async_copy(
                    t_hbm.at[f].at[idx_v.at[pl.ds(
                        f_loc * W + jj * 2 * B_CHK + s * B_CHK, B_CHK)]],
                    gbufs.at[gslot + s], gsem)

        def write_start(p, wbase):
            f = p // NPAIR
            jj = p % NPAIR
            for h in range(2):
                pltpu.async_copy(
                    wbufs.at[wbase + h],
                    out_hbm.at[pl.ds(f * EMB + h * B_CHK, B_CHK),
                               pl.ds(base + jj * 2 * B_CHK, 2 * B_CHK)], wsem)

        def gather_wait():
            pltpu.make_async_copy(t_hbm.at[0, pl.ds(0, B_CHK)], gbufs.at[0],
                                  gsem).wait()

        def write_wait():
            pltpu.make_async_copy(gbufs.at[0],
                                  out_hbm.at[pl.ds(0, B_CHK),
                                             pl.ds(0, 2 * B_CHK)],
                                  wsem).wait()

        def transpose_pair(gslot, wbase):
            @pl.loop(0, L)
            def _(d):
                dm = dtab[d, pl.ds(0, L)]
                for h in range(2):
                    for s in range(2):
                        for eq in range(4):
                            cols_l = dm + (h * B_CHK + eq * L)
                            rows_w = dm + (eq * L)
                            vs = [plsc.load_gather(gbufs.at[gslot + s],
                                                   [cvec[bq], cols_l])
                                  for bq in range(4)]
                            for bq in range(4):
                                plsc.store_scatter(
                                    wbufs.at[wbase + h],
                                    [rows_w, cvec[s * 4 + bq]], vs[bq])

        load_idx_half(0)
        gather_start(0, 0)

        @pl.loop(0, TOTALP // 2)
        def _task(u):
            for par in range(2):
                p = u * 2 + par
                gslot = 2 * par
                f = p // NPAIR
                jj = p % NPAIR
                f_loc = lax.rem(f, FH)
                chunks = [idx_v[pl.ds(f_loc * W + jj * 2 * B_CHK + c * L, L)]
                          for c in range(8)]

                @pl.when(p + 1 < TOTALP)
                def _():
                    @pl.when(p + 1 == (TOTALP // 2))
                    def _():
                        load_idx_half(1)
                    gather_start(p + 1, 2 - gslot)

                gather_wait()
                gather_wait()

                @pl.when(p >= 2)
                def _():
                    write_wait()
                    write_wait()

                transpose_pair(gslot, gslot)

                for c in range(8):
                    chunk = chunks[c]

                    @pl.when(jnp.min(chunk) == 0)
                    def _():
                        m = chunk == 0

                        @pl.loop(0, B_CHK)
                        def _(e):
                            er = jnp.full((L,), e, jnp.int32)
                            for h in range(2):
                                plsc.store_scatter(wbufs.at[gslot + h],
                                                   [er, cvec[c]], zeros16,
                                                   mask=m)

                write_start(p, gslot)

        write_wait()
        write_wait()
        write_wait()
        write_wait()

        pltpu.sync_copy(dense_hbm.at[:, pl.ds(base, W)],
                        out_hbm.at[pl.ds(N_FIELDS * EMB, DENSE_DIM),
                                   pl.ds(base, W)])

    return k(feats_flat, dense_t_hbm, t)


def kernel(feat_0, feat_1, feat_2, feat_3, feat_4, feat_5, feat_6, feat_7,
           feat_8, feat_9, feat_10, feat_11, feat_12, feat_13, feat_14,
           feat_15, feat_16, feat_17, feat_18, feat_19, feat_20, feat_21,
           feat_22, feat_23, feat_24, feat_25, dense, tables):
    feats_flat = jnp.concatenate([
        feat_0, feat_1, feat_2, feat_3, feat_4, feat_5, feat_6, feat_7,
        feat_8, feat_9, feat_10, feat_11, feat_12, feat_13, feat_14, feat_15,
        feat_16, feat_17, feat_18, feat_19, feat_20, feat_21, feat_22,
        feat_23, feat_24, feat_25,
    ]).astype(jnp.int32)
    dense_t = _dense_transpose(dense.astype(jnp.float32))
    out_t = _sc_embed_t(feats_flat, dense_t, tables.astype(jnp.float32))
    return jnp.transpose(out_t)


# speedup vs baseline: 4.1649x; 1.0082x over previous
import functools

import jax
import jax.numpy as jnp
from jax import lax
from jax.experimental import pallas as pl
from jax.experimental.pallas import tpu as pltpu
from jax.experimental.pallas import tpu_sc as plsc

N_FIELDS = 26
VOCAB_P1 = 1001
EMB = 128
BATCH = 16384
DENSE_DIM = 13
OUT_DIM = N_FIELDS * EMB + DENSE_DIM

NC, NS = 2, 16
NW = NC * NS
W = BATCH // NW
B_CHK = 64
NPAIR = W // (2 * B_CHK)
TOTALP = N_FIELDS * NPAIR
L = 16
FH = N_FIELDS // 2
DH = W // 2


def _dense_transpose(dense):
    def body(d_ref, o_ref):
        o_ref[...] = jnp.transpose(d_ref[...], (1, 0))

    return pl.pallas_call(
        body,
        grid=(8,),
        in_specs=[pl.BlockSpec((BATCH // 8, DENSE_DIM), lambda i: (i, 0))],
        out_specs=pl.BlockSpec((DENSE_DIM, BATCH // 8), lambda i: (0, i)),
        out_shape=jax.ShapeDtypeStruct((DENSE_DIM, BATCH), jnp.float32),
    )(dense)


def _sc_embed_t(feats, dense_t_hbm, t):
    mesh = plsc.VectorSubcoreMesh(core_axis_name="c", subcore_axis_name="s")

    @functools.partial(
        pl.kernel,
        out_type=jax.ShapeDtypeStruct((OUT_DIM, BATCH), jnp.float32),
        mesh=mesh,
        scratch_types=[
            pltpu.VMEM((FH * W,), jnp.int32),
            pltpu.VMEM((4, B_CHK, EMB), jnp.float32),
            pltpu.VMEM((4, B_CHK, 2 * B_CHK), jnp.float32),
            pltpu.VMEM((L, L), jnp.int32),
            pltpu.SemaphoreType.DMA,
            pltpu.SemaphoreType.DMA,
            pltpu.SemaphoreType.DMA,
        ],
        compiler_params=pltpu.CompilerParams(needs_layout_passes=False),
    )
    def k(*refs):
        feats_hbm = refs[:N_FIELDS]
        (dense_hbm, t_hbm, out_hbm, idx_v, gbufs, wbufs,
         dtab, gsem, wsem, isem) = refs[N_FIELDS:]
        wid = lax.axis_index("c") * NS + lax.axis_index("s")
        base = wid * W

        iota = lax.broadcasted_iota(jnp.int32, (L,), 0)
        for d in range(L):
            dtab[d, pl.ds(0, L)] = lax.rem(iota + d, L)
        cvec = [iota + q * L for q in range(8)]
        zeros16 = jnp.zeros((L,), jnp.float32)

        def load_idx_half(h):
            for fl in range(FH):
                pltpu.async_copy(
                    feats_hbm[h * FH + fl].at[pl.ds(base, W)],
                    idx_v.at[pl.ds(fl * W, W)], isem)
            for fl in range(FH):
                pltpu.make_async_copy(feats_hbm[0].at[pl.ds(0, W)],
                                      idx_v.at[pl.ds(0, W)], isem).wait()

        def gather_start(p, gslot):
            f = p // NPAIR
            jj = p % NPAIR
            f_loc = lax.rem(f, FH)
            for s in range(2):
                pltpu.async_copy(
                    t_hbm.at[f].at[idx_v.at[pl.ds(
                        f_loc * W + jj * 2 * B_CHK + s * B_CHK, B_CHK)]],
                    gbufs.at[gslot + s], gsem)

        def write_start(p, wbase):
            f = p // NPAIR
            jj = p % NPAIR
            for h in range(2):
                pltpu.async_copy(
                    wbufs.at[wbase + h],
                    out_hbm.at[pl.ds(f * EMB + h * B_CHK, B_CHK),
                               pl.ds(base + jj * 2 * B_CHK, 2 * B_CHK)], wsem)

        def gather_wait():
            pltpu.make_async_copy(t_hbm.at[0, pl.ds(0, B_CHK)], gbufs.at[0],
                                  gsem).wait()

        def write_wait():
            pltpu.make_async_copy(gbufs.at[0],
                                  out_hbm.at[pl.ds(0, B_CHK),
                                             pl.ds(0, 2 * B_CHK)],
                                  wsem).wait()

        def transpose_pair(gslot, wbase):
            @pl.loop(0, L)
            def _(d):
                dm = dtab[d, pl.ds(0, L)]
                for h in range(2):
                    for s in range(2):
                        for eq in range(4):
                            cols_l = dm + (h * B_CHK + eq * L)
                            rows_w = dm + (eq * L)
                            vs = [plsc.load_gather(gbufs.at[gslot + s],
                                                   [cvec[bq], cols_l])
                                  for bq in range(4)]
                            for bq in range(4):
                                plsc.store_scatter(
                                    wbufs.at[wbase + h],
                                    [rows_w, cvec[s * 4 + bq]], vs[bq])

        load_idx_half(0)
        gather_start(0, 0)

        @pl.loop(0, TOTALP // 2)
        def _task(u):
            for par in range(2):
                p = u * 2 + par
                gslot = 2 * par
                f = p // NPAIR
                jj = p % NPAIR
                f_loc = lax.rem(f, FH)
                chunks = [idx_v[pl.ds(f_loc * W + jj * 2 * B_CHK + c * L, L)]
                          for c in range(8)]

                @pl.when(p + 1 < TOTALP)
                def _():
                    @pl.when(p + 1 == (TOTALP // 2))
                    def _():
                        load_idx_half(1)
                    gather_start(p + 1, 2 - gslot)

                gather_wait()
                gather_wait()

                @pl.when(p >= 2)
                def _():
                    write_wait()
                    write_wait()

                transpose_pair(gslot, gslot)

                for c in range(8):
                    chunk = chunks[c]

                    @pl.when(jnp.min(chunk) == 0)
                    def _():
                        m = chunk == 0

                        @pl.loop(0, B_CHK)
                        def _(e):
                            er = jnp.full((L,), e, jnp.int32)
                            for h in range(2):
                                plsc.store_scatter(wbufs.at[gslot + h],
                                                   [er, cvec[c]], zeros16,
                                                   mask=m)

                write_start(p, gslot)

        write_wait()
        write_wait()
        write_wait()
        write_wait()

        pltpu.sync_copy(dense_hbm.at[:, pl.ds(base, W)],
                        out_hbm.at[pl.ds(N_FIELDS * EMB, DENSE_DIM),
                                   pl.ds(base, W)])

    return k(*feats, dense_t_hbm, t)


def kernel(feat_0, feat_1, feat_2, feat_3, feat_4, feat_5, feat_6, feat_7,
           feat_8, feat_9, feat_10, feat_11, feat_12, feat_13, feat_14,
           feat_15, feat_16, feat_17, feat_18, feat_19, feat_20, feat_21,
           feat_22, feat_23, feat_24, feat_25, dense, tables):
    feats = tuple(f.astype(jnp.int32) for f in (
        feat_0, feat_1, feat_2, feat_3, feat_4, feat_5, feat_6, feat_7,
        feat_8, feat_9, feat_10, feat_11, feat_12, feat_13, feat_14, feat_15,
        feat_16, feat_17, feat_18, feat_19, feat_20, feat_21, feat_22,
        feat_23, feat_24, feat_25,
    ))
    dense_t = _dense_transpose(dense.astype(jnp.float32))
    out_t = _sc_embed_t(feats, dense_t, tables.astype(jnp.float32))
    return jnp.transpose(out_t)


# speedup vs baseline: 4.2708x; 1.0254x over previous
import functools

import jax
import jax.numpy as jnp
from jax import lax
from jax.experimental import pallas as pl
from jax.experimental.pallas import tpu as pltpu
from jax.experimental.pallas import tpu_sc as plsc

N_FIELDS = 26
VOCAB_P1 = 1001
EMB = 128
BATCH = 16384
DENSE_DIM = 13
OUT_DIM = N_FIELDS * EMB + DENSE_DIM

NC, NS = 2, 16
NW = NC * NS
W = BATCH // NW
B_CHK = 64
NPAIR = W // (2 * B_CHK)
TOTALP = N_FIELDS * NPAIR
L = 16
FH = N_FIELDS // 2
DH = W // 2


def _dense_transpose(dense):
    def body(d_ref, o_ref):
        o_ref[...] = jnp.transpose(d_ref[...], (1, 0))

    return pl.pallas_call(
        body,
        grid=(8,),
        in_specs=[pl.BlockSpec((BATCH // 8, DENSE_DIM), lambda i: (i, 0))],
        out_specs=pl.BlockSpec((DENSE_DIM, BATCH // 8), lambda i: (0, i)),
        out_shape=jax.ShapeDtypeStruct((DENSE_DIM, BATCH), jnp.float32),
    )(dense)


def _sc_embed_t(feats, dense_t_hbm, t):
    mesh = plsc.VectorSubcoreMesh(core_axis_name="c", subcore_axis_name="s")

    @functools.partial(
        pl.kernel,
        out_type=jax.ShapeDtypeStruct((OUT_DIM, BATCH), jnp.float32),
        mesh=mesh,
        scratch_types=[
            pltpu.VMEM((FH * W,), jnp.int32),
            pltpu.VMEM((4, B_CHK, EMB), jnp.float32),
            pltpu.VMEM((4, B_CHK, 2 * B_CHK), jnp.float32),
            pltpu.VMEM((L, L), jnp.int32),
            pltpu.SemaphoreType.DMA,
            pltpu.SemaphoreType.DMA,
            pltpu.SemaphoreType.DMA,
        ],
        compiler_params=pltpu.CompilerParams(needs_layout_passes=False),
    )
    def k(*refs):
        feats_hbm = refs[:N_FIELDS]
        (dense_hbm, t_hbm, out_hbm, idx_v, gbufs, wbufs,
         dtab, gsem, wsem, isem) = refs[N_FIELDS:]
        wid = lax.axis_index("c") * NS + lax.axis_index("s")
        base = wid * W

        iota = lax.broadcasted_iota(jnp.int32, (L,), 0)
        for d in range(L):
            dtab[d, pl.ds(0, L)] = lax.rem(iota + d, L)
        cvec = [iota + q * L for q in range(8)]
        zeros16 = jnp.zeros((L,), jnp.float32)

        def load_idx_half(h):
            for fl in range(FH):
                pltpu.async_copy(
                    feats_hbm[h * FH + fl].at[pl.ds(base, W)],
                    idx_v.at[pl.ds(fl * W, W)], isem)
            for fl in range(FH):
                pltpu.make_async_copy(feats_hbm[0].at[pl.ds(0, W)],
                                      idx_v.at[pl.ds(0, W)], isem).wait()

        def gather_start(p, gslot):
            f = p // NPAIR
            jj = p % NPAIR
            f_loc = lax.rem(f, FH)
            for s in range(2):
                pltpu.async_copy(
                    t_hbm.at[f].at[idx_v.at[pl.ds(
                        f_loc * W + jj * 2 * B_CHK + s * B_CHK, B_CHK)]],
                    gbufs.at[gslot + s], gsem)

        def write_start(p, wbase):
            f = p // NPAIR
            jj = p % NPAIR
            for h in range(2):
                pltpu.async_copy(
                    wbufs.at[wbase + h],
                    out_hbm.at[pl.ds(f * EMB + h * B_CHK, B_CHK),
                               pl.ds(base + jj * 2 * B_CHK, 2 * B_CHK)], wsem)

        def gather_wait():
            pltpu.make_async_copy(t_hbm.at[0, pl.ds(0, B_CHK)], gbufs.at[0],
                                  gsem).wait()

        def write_wait():
            pltpu.make_async_copy(gbufs.at[0],
                                  out_hbm.at[pl.ds(0, B_CHK),
                                             pl.ds(0, 2 * B_CHK)],
                                  wsem).wait()

        def transpose_pair(gslot, wbase):
            @pl.loop(0, L, step=2)
            def _(d):
                for dd in range(2):
                    dm = dtab[d + dd, pl.ds(0, L)]
                    for h in range(2):
                        for s in range(2):
                            for eq in range(4):
                                cols_l = dm + (h * B_CHK + eq * L)
                                rows_w = dm + (eq * L)
                                vs = [plsc.load_gather(gbufs.at[gslot + s],
                                                       [cvec[bq], cols_l])
                                      for bq in range(4)]
                                for bq in range(4):
                                    plsc.store_scatter(
                                        wbufs.at[wbase + h],
                                        [rows_w, cvec[s * 4 + bq]], vs[bq])

        load_idx_half(0)
        gather_start(0, 0)

        @pl.loop(0, TOTALP // 2)
        def _task(u):
            for par in range(2):
                p = u * 2 + par
                gslot = 2 * par
                f = p // NPAIR
                jj = p % NPAIR
                f_loc = lax.rem(f, FH)
                chunks = [idx_v[pl.ds(f_loc * W + jj * 2 * B_CHK + c * L, L)]
                          for c in range(8)]

                @pl.when(p + 1 < TOTALP)
                def _():
                    @pl.when(p + 1 == (TOTALP // 2))
                    def _():
                        load_idx_half(1)
                    gather_start(p + 1, 2 - gslot)

                gather_wait()
                gather_wait()

                @pl.when(p >= 2)
                def _():
                    write_wait()
                    write_wait()

                transpose_pair(gslot, gslot)

                for c in range(8):
                    chunk = chunks[c]

                    @pl.when(jnp.min(chunk) == 0)
                    def _():
                        m = chunk == 0

                        @pl.loop(0, B_CHK)
                        def _(e):
                            er = jnp.full((L,), e, jnp.int32)
                            for h in range(2):
                                plsc.store_scatter(wbufs.at[gslot + h],
                                                   [er, cvec[c]], zeros16,
                                                   mask=m)

                write_start(p, gslot)

        write_wait()
        write_wait()
        write_wait()
        write_wait()

        pltpu.sync_copy(dense_hbm.at[:, pl.ds(base, W)],
                        out_hbm.at[pl.ds(N_FIELDS * EMB, DENSE_DIM),
                                   pl.ds(base, W)])

    return k(*feats, dense_t_hbm, t)


def kernel(feat_0, feat_1, feat_2, feat_3, feat_4, feat_5, feat_6, feat_7,
           feat_8, feat_9, feat_10, feat_11, feat_12, feat_13, feat_14,
           feat_15, feat_16, feat_17, feat_18, feat_19, feat_20, feat_21,
           feat_22, feat_23, feat_24, feat_25, dense, tables):
    feats = tuple(f.astype(jnp.int32) for f in (
        feat_0, feat_1, feat_2, feat_3, feat_4, feat_5, feat_6, feat_7,
        feat_8, feat_9, feat_10, feat_11, feat_12, feat_13, feat_14, feat_15,
        feat_16, feat_17, feat_18, feat_19, feat_20, feat_21, feat_22,
        feat_23, feat_24, feat_25,
    ))
    dense_t = _dense_transpose(dense.astype(jnp.float32))
    out_t = _sc_embed_t(feats, dense_t, tables.astype(jnp.float32))
    return jnp.transpose(out_t)
